# Initial kernel scaffold; baseline (speedup 1.0000x reference)
import jax
import jax.numpy as jnp
from jax.experimental import pallas as pl


def kernel(x, edge_index, edge_attr, params):
    raise NotImplementedError("write your pallas kernel here")


# speedup vs baseline: 2.8629x; 2.8629x over previous
import functools
from typing import Sequence

import jax
import jax.numpy as jnp
from jax import lax
from jax.experimental import pallas as pl
from jax.experimental.pallas import tpu as pltpu
from jax.experimental.pallas import tpu_sc as plsc

N = 10000
NPAD = 10240
E = 320000
IN_DIM = 128
HID = 64
LAYERS = 3
RBF_K = 16
NCLS = 26
WY_START = 100
WY_DIM = 26

BN = 1024
BE = 2560
NW = 32
EW = E // NW
GC = 200
SC_CH = 1000
ROWS_PER_TILE = NPAD // 16


def _silu(x):
    return x * (1.0 / (1.0 + jnp.exp(-x)))


def _ln(x, g, b, eps=1e-5):
    mu = jnp.mean(x, axis=-1, keepdims=True)
    var = jnp.mean((x - mu) ** 2, axis=-1, keepdims=True)
    return (x - mu) / jnp.sqrt(var + eps) * g + b


def _encode_body(x_ref, w_in_ref, b_in_ref, g_ref, bb_ref, wab_ref,
                 h_ref, ab_ref):
    t = jnp.dot(x_ref[...], w_in_ref[...],
                preferred_element_type=jnp.float32) + b_in_ref[...]
    h = _silu(_ln(t, g_ref[...], bb_ref[...]))
    h_ref[...] = h
    ab_ref[...] = jnp.dot(h, wab_ref[...], preferred_element_type=jnp.float32)


def _encode(x, w_in, b_in, g, b, wab):
    grid = NPAD // BN
    fullw = lambda s: pl.BlockSpec(s, lambda i: (0, 0))
    return pl.pallas_call(
        _encode_body,
        grid=(grid,),
        in_specs=[
            pl.BlockSpec((BN, IN_DIM), lambda i: (i, 0)),
            fullw((IN_DIM, HID)), fullw((1, HID)), fullw((1, HID)),
            fullw((1, HID)), fullw((HID, 2 * HID)),
        ],
        out_specs=[pl.BlockSpec((BN, HID), lambda i: (i, 0)),
                   pl.BlockSpec((BN, 2 * HID), lambda i: (i, 0))],
        out_shape=[jax.ShapeDtypeStruct((NPAD, HID), jnp.float32),
                   jax.ShapeDtypeStruct((NPAD, 2 * HID), jnp.float32)],
    )(x, w_in, b_in, g, b, wab)


def _quantile_body(eat_ref, out_ref):
    v = eat_ref[...]
    r2 = v[0:1] ** 2 + v[1:2] ** 2 + v[2:3] ** 2
    r = jnp.maximum(jnp.sqrt(r2), 1e-8)
    rbits = lax.bitcast_convert_type(r, jnp.int32)
    lo0 = jnp.min(rbits)
    hi0 = jnp.max(rbits)

    def orderstat(k):
        def body(_, carry):
            lo, hi = carry
            mid = lo + (hi - lo) // 2
            midf = lax.bitcast_convert_type(mid, jnp.float32)
            cnt = jnp.sum((r <= midf).astype(jnp.int32))
            ge = cnt >= (k + 1)
            return (jnp.where(ge, lo, mid + 1), jnp.where(ge, mid, hi))
        lo, hi = lax.fori_loop(0, 32, body, (lo0, hi0))
        return lax.bitcast_convert_type(hi, jnp.float32)

    q_pos = 0.95 * (E - 1)
    k_lo = int(q_pos)
    frac = jnp.float32(q_pos - k_lo)
    r1 = orderstat(k_lo)
    r2s = orderstat(k_lo + 1)
    q = r1 * (1.0 - frac) + r2s * frac
    out_ref[0, 0] = jnp.clip(q, 1.0, 8.0)


def _quantile(ea_t):
    return pl.pallas_call(
        _quantile_body,
        in_specs=[pl.BlockSpec((4, E), lambda: (0, 0))],
        out_specs=pl.BlockSpec((1, 1), lambda: (0, 0), memory_space=pltpu.SMEM),
        out_shape=jax.ShapeDtypeStruct((1, 1), jnp.float32),
    )(ea_t)


def _edge_enc_body(eat_ref, rmax_ref, w_e_ref, b_e_ref, g_ref, bb_ref, out_ref):
    v = eat_ref[...]
    r = jnp.maximum(jnp.sqrt(v[0:1] ** 2 + v[1:2] ** 2 + v[2:3] ** 2), 1e-8)
    u = v[0:3] / r
    r_max = rmax_ref[0, 0]
    delta = jnp.maximum(r_max / (RBF_K - 1), 1e-3)
    gamma = 1.0 / (2.0 * (0.5 * delta) ** 2)
    kk = lax.broadcasted_iota(jnp.int32, (RBF_K, BE), 0).astype(jnp.float32)
    centers = r_max * kk / (RBF_K - 1)
    rbf = jnp.exp(-gamma * (r - centers) ** 2)
    e_t = jnp.concatenate([u, r, rbf], axis=0)
    t = lax.dot_general(e_t, w_e_ref[...], (((0,), (0,)), ((), ())),
                        preferred_element_type=jnp.float32) + b_e_ref[...]
    out_ref[...] = _ln(_silu(t), g_ref[...], bb_ref[...])


def _edge_enc(ea_t, r_max, w_e, b_e, g, b):
    fullw = lambda s: pl.BlockSpec(s, lambda i: (0, 0))
    return pl.pallas_call(
        _edge_enc_body,
        grid=(E // BE,),
        in_specs=[
            pl.BlockSpec((4, BE), lambda i: (0, i)),
            pl.BlockSpec(memory_space=pltpu.SMEM),
            fullw((3 + 1 + RBF_K, HID)), fullw((1, HID)),
            fullw((1, HID)), fullw((1, HID)),
        ],
        out_specs=pl.BlockSpec((BE, HID), lambda i: (i, 0)),
        out_shape=jax.ShapeDtypeStruct((E, HID), jnp.float32),
    )(ea_t, r_max, w_e, b_e, g, b)


def _message_body(gath_ref, eenc_ref, w1c_ref, b1_ref, w2_ref, b2t_ref, out_ref):
    pre = gath_ref[...] + jnp.dot(eenc_ref[...], w1c_ref[...],
                                  preferred_element_type=jnp.float32) + b1_ref[...]
    t = _silu(pre)
    m2t = lax.dot_general(w2_ref[...], t, (((0,), (1,)), ((), ())),
                          preferred_element_type=jnp.float32)
    out_ref[...] = _silu(m2t + b2t_ref[...])


def _message(gathered, e_enc, w1c, b1, w2, b2t):
    fullw = lambda s: pl.BlockSpec(s, lambda i: (0, 0))
    return pl.pallas_call(
        _message_body,
        grid=(E // BE,),
        in_specs=[
            pl.BlockSpec((BE, HID), lambda i: (i, 0)),
            pl.BlockSpec((BE, HID), lambda i: (i, 0)),
            fullw((HID, HID)), fullw((1, HID)),
            fullw((HID, HID)), fullw((HID, 1)),
        ],
        out_specs=pl.BlockSpec((HID, BE), lambda i: (0, i)),
        out_shape=jax.ShapeDtypeStruct((HID, E), jnp.float32),
    )(gathered, e_enc, w1c, b1, w2, b2t)


def _update_body(h_ref, ms_ref, wuh_ref, wum_ref, bu_ref, g_ref, bb_ref,
                 wab_ref, h_out, ab_out):
    h = h_ref[...]
    ms = ms_ref[...]
    m_t = ms[0] + ms[1] + ms[2] + ms[3]
    t = (jnp.dot(h, wuh_ref[...], preferred_element_type=jnp.float32)
         + lax.dot_general(m_t, wum_ref[...], (((0,), (0,)), ((), ())),
                           preferred_element_type=jnp.float32)
         + bu_ref[...])
    hn = _ln(_silu(t) + h, g_ref[...], bb_ref[...])
    h_out[...] = hn
    if ab_out is not None:
        ab_out[...] = jnp.dot(hn, wab_ref[...],
                              preferred_element_type=jnp.float32)


def _update(h, msums, wuh, wum, bu, g, b, wab, with_ab):
    fullw = lambda s: pl.BlockSpec(s, lambda i: (0, 0))
    body = _update_body if with_ab else (
        lambda h_ref, ms_ref, wuh_ref, wum_ref, bu_ref, g_ref, bb_ref,
               wab_ref, h_out:
        _update_body(h_ref, ms_ref, wuh_ref, wum_ref, bu_ref, g_ref, bb_ref,
                     wab_ref, h_out, None))
    out_specs = [pl.BlockSpec((BN, HID), lambda i: (i, 0))]
    out_shape = [jax.ShapeDtypeStruct((NPAD, HID), jnp.float32)]
    if with_ab:
        out_specs.append(pl.BlockSpec((BN, 2 * HID), lambda i: (i, 0)))
        out_shape.append(jax.ShapeDtypeStruct((NPAD, 2 * HID), jnp.float32))
    out = pl.pallas_call(
        body,
        grid=(NPAD // BN,),
        in_specs=[
            pl.BlockSpec((BN, HID), lambda i: (i, 0)),
            pl.BlockSpec((4, HID, BN), lambda i: (0, 0, i)),
            fullw((HID, HID)), fullw((HID, HID)), fullw((1, HID)),
            fullw((1, HID)), fullw((1, HID)),
            fullw((HID, 2 * HID)),
        ],
        out_specs=out_specs,
        out_shape=out_shape,
    )(h, msums, wuh, wum, bu, g, b, wab)
    return out if with_ab else (out[0], None)


def _head_body(h0_ref, h1_ref, h2_ref, h3_ref, wjk0, wjk1, wjk2, wjk3,
               bjk_ref, wo1_ref, bo1_ref, wo2_ref, bo2_ref, out_ref):
    agg = (jnp.dot(h0_ref[...], wjk0[...], preferred_element_type=jnp.float32)
           + jnp.dot(h1_ref[...], wjk1[...], preferred_element_type=jnp.float32)
           + jnp.dot(h2_ref[...], wjk2[...], preferred_element_type=jnp.float32)
           + jnp.dot(h3_ref[...], wjk3[...], preferred_element_type=jnp.float32)
           + bjk_ref[...])
    z = _silu(jnp.dot(agg, wo1_ref[...],
                      preferred_element_type=jnp.float32) + bo1_ref[...])
    out_ref[...] = jnp.dot(z, wo2_ref[...],
                           preferred_element_type=jnp.float32) + bo2_ref[...]


def _head(hs, wjks, bjk, wo1, bo1, wo2, bo2):
    fullw = lambda s: pl.BlockSpec(s, lambda i: (0, 0))
    return pl.pallas_call(
        _head_body,
        grid=(NPAD // BN,),
        in_specs=(
            [pl.BlockSpec((BN, HID), lambda i: (i, 0))] * 4
            + [fullw((HID, HID))] * 4
            + [fullw((1, HID)), fullw((HID, HID)), fullw((1, HID)),
               fullw((HID, NCLS)), fullw((1, NCLS))]
        ),
        out_specs=pl.BlockSpec((BN, NCLS), lambda i: (i, 0)),
        out_shape=jax.ShapeDtypeStruct((NPAD, NCLS), jnp.float32),
    )(*hs, *wjks, bjk, wo1, bo1, wo2, bo2)


def _pool_body(xw_ref, lg_ref, out_ref):
    xw = xw_ref[...]
    logits = lg_ref[...]
    rowmax = jnp.max(xw, axis=1, keepdims=True)
    colidx = lax.broadcasted_iota(jnp.int32, (NPAD, NCLS), 1)
    cand = jnp.where(xw == rowmax, colidx, NCLS)
    gid = jnp.min(cand, axis=1, keepdims=True)
    valid = lax.broadcasted_iota(jnp.int32, (NPAD, 1), 0) < N
    onehot = jnp.where((colidx == gid) & valid, 1.0, 0.0)
    sums = lax.dot_general(onehot, logits, (((0,), (0,)), ((), ())),
                           preferred_element_type=jnp.float32)
    counts = jnp.sum(onehot, axis=0, keepdims=True)
    invc = 1.0 / jnp.maximum(counts, 1.0)
    node_ms = jnp.dot(onehot, sums, preferred_element_type=jnp.float32)
    node_inv = jnp.sum(onehot * invc, axis=1, keepdims=True)
    out_ref[...] = node_ms * node_inv


def _pool(xw, logits):
    return pl.pallas_call(
        _pool_body,
        in_specs=[pl.BlockSpec((NPAD, NCLS), lambda: (0, 0))] * 2,
        out_specs=pl.BlockSpec((NPAD, NCLS), lambda: (0, 0)),
        out_shape=jax.ShapeDtypeStruct((NPAD, NCLS), jnp.float32),
    )(xw, logits)


_MESH = dict(core_axis_name="c", subcore_axis_name="s")


def _gather_add_sc(ab, src, dst):
    mesh = plsc.VectorSubcoreMesh(**_MESH)

    @functools.partial(
        pl.kernel,
        out_type=jax.ShapeDtypeStruct((E, HID), jnp.float32),
        mesh=mesh,
        scratch_types=[
            pltpu.VMEM((GC,), jnp.int32),
            pltpu.VMEM((GC,), jnp.int32),
            pltpu.VMEM((GC, 2 * HID), jnp.float32),
            pltpu.VMEM((GC, 2 * HID), jnp.float32),
            pltpu.VMEM((GC, HID), jnp.float32),
            pltpu.SemaphoreType.DMA,
            pltpu.SemaphoreType.DMA,
        ],
    )
    def k(ab_hbm, src_hbm, dst_hbm, out_hbm,
          sidx, didx, arows, brows, obuf, sem1, sem2):
        wid = lax.axis_index("s") * 2 + lax.axis_index("c")

        def chunk(ch, _):
            base = wid * EW + ch * GC
            pltpu.sync_copy(src_hbm.at[pl.ds(base, GC)], sidx)
            pltpu.sync_copy(dst_hbm.at[pl.ds(base, GC)], didx)
            cp_a = pltpu.async_copy(ab_hbm.at[sidx], arows, sem1)
            cp_b = pltpu.async_copy(ab_hbm.at[didx], brows, sem2)
            cp_a.wait()
            cp_b.wait()

            def row(i, _):
                for q in range(HID // 16):
                    sl = pl.ds(q * 16, 16)
                    obuf[i, sl] = arows[i, sl] + brows[i, pl.ds(HID + q * 16, 16)]
                return 0
            lax.fori_loop(0, GC, row, 0)
            pltpu.sync_copy(obuf, out_hbm.at[pl.ds(base, GC)])
            return 0

        lax.fori_loop(0, EW // GC, chunk, 0)

    return k(ab, src, dst)


NG = 4
NFS = 8
EG = E // NG
SCH = 3200


def _scatter_add_sc(m2t, dst):
    mesh = plsc.VectorSubcoreMesh(**_MESH)

    @functools.partial(
        pl.kernel,
        out_type=jax.ShapeDtypeStruct((NG, HID, NPAD), jnp.float32),
        mesh=mesh,
        scratch_types=[
            pltpu.VMEM((SCH,), jnp.int32),
            pltpu.VMEM((NFS, SCH), jnp.float32),
            pltpu.VMEM((NFS * NPAD,), jnp.float32),
        ],
        compiler_params=pltpu.CompilerParams(needs_layout_passes=False),
    )
    def k(m2t_hbm, dst_hbm, out_hbm, didx, rows, acc):
        wid = lax.axis_index("s") * 2 + lax.axis_index("c")
        fs = wid % NFS
        g = wid // NFS

        zero16 = jnp.zeros((16,), jnp.float32)

        def zrow(i, _):
            acc[pl.ds(i * 16, 16)] = zero16
            return 0
        lax.fori_loop(0, NFS * NPAD // 16, zrow, 0)

        def chunk(ch, _):
            base = g * EG + ch * SCH
            pltpu.sync_copy(dst_hbm.at[pl.ds(base, SCH)], didx)
            pltpu.sync_copy(
                m2t_hbm.at[pl.ds(fs * NFS, NFS), pl.ds(base, SCH)], rows)

            full = jnp.ones((16,), jnp.bool_)

            def vec(j, _):
                dstv = didx[pl.ds(j * 16, 16)]
                for f in range(NFS):
                    vals = rows[f, pl.ds(j * 16, 16)]
                    plsc.addupdate_scatter(acc, [dstv + f * NPAD], vals,
                                           mask=full)
                return 0
            lax.fori_loop(0, SCH // 16, vec, 0)
            return 0
        lax.fori_loop(0, EG // SCH, chunk, 0)

        for f in range(NFS):
            pltpu.sync_copy(acc.at[pl.ds(f * NPAD, NPAD)],
                            out_hbm.at[g, fs * NFS + f])

    return k(m2t, dst)


def kernel(x, edge_index, edge_attr, params):
    p = params
    xp = jnp.pad(x, ((0, NPAD - N), (0, 0)))
    ea_t = edge_attr.T
    src = edge_index[0]
    dst = edge_index[1]

    row = lambda v: v.reshape(1, -1)

    w1ab = [p['msg_W1'][l][:2 * HID] for l in range(LAYERS)]
    w1c = [p['msg_W1'][l][2 * HID:] for l in range(LAYERS)]
    wab = [jnp.concatenate([w[:HID], w[HID:]], axis=1) for w in w1ab]

    h, ab = _encode(xp, p['W_in'], row(p['b_in']), row(p['ln_in_g']),
                    row(p['ln_in_b']), wab[0])

    r_max = _quantile(ea_t)
    e_enc = _edge_enc(ea_t, r_max, p['W_e'], row(p['b_e']),
                      row(p['ln_e_g']), row(p['ln_e_b']))

    states = [h]
    for l in range(LAYERS):
        gathered = _gather_add_sc(ab, src, dst)
        m2 = _message(gathered, e_enc, w1c[l], row(p['msg_b1'][l]),
                      p['msg_W2'][l], p['msg_b2'][l].reshape(-1, 1))
        msums = _scatter_add_sc(m2, dst)
        with_ab = l < LAYERS - 1
        wab_n = wab[l + 1] if with_ab else wab[0]
        h, ab = _update(h, msums, p['upd_W'][l][:HID], p['upd_W'][l][HID:],
                        row(p['upd_b'][l]), row(p['ln_g'][l]),
                        row(p['ln_b'][l]), wab_n, with_ab)
        states.append(h)

    wjks = [p['W_jk'][l * HID:(l + 1) * HID] for l in range(LAYERS + 1)]
    logits = _head(states, wjks, row(p['b_jk']), p['W_o1'], row(p['b_o1']),
                   p['W_o2'], row(p['b_o2']))

    xw = xp[:, WY_START:WY_START + WY_DIM]
    out = _pool(xw, logits)
    return out[:N]


# speedup vs baseline: 2.8704x; 1.0026x over previous
import functools
from typing import Sequence

import jax
import jax.numpy as jnp
from jax import lax
from jax.experimental import pallas as pl
from jax.experimental.pallas import tpu as pltpu
from jax.experimental.pallas import tpu_sc as plsc

N = 10000
NPAD = 10240
E = 320000
IN_DIM = 128
HID = 64
LAYERS = 3
RBF_K = 16
NCLS = 26
WY_START = 100
WY_DIM = 26

BN = 1024
BE = 2560
NW = 32
EW = E // NW
GC = 200
SC_CH = 1000
ROWS_PER_TILE = NPAD // 16


def _silu(x):
    return x * (1.0 / (1.0 + jnp.exp(-x)))


def _ln(x, g, b, eps=1e-5):
    mu = jnp.mean(x, axis=-1, keepdims=True)
    var = jnp.mean((x - mu) ** 2, axis=-1, keepdims=True)
    return (x - mu) / jnp.sqrt(var + eps) * g + b


def _encode_body(x_ref, w_in_ref, b_in_ref, g_ref, bb_ref, wab_ref,
                 h_ref, ab_ref):
    t = jnp.dot(x_ref[...], w_in_ref[...],
                preferred_element_type=jnp.float32) + b_in_ref[...]
    h = _silu(_ln(t, g_ref[...], bb_ref[...]))
    h_ref[...] = h
    ab_ref[...] = jnp.dot(h, wab_ref[...], preferred_element_type=jnp.float32)


def _encode(x, w_in, b_in, g, b, wab):
    grid = NPAD // BN
    fullw = lambda s: pl.BlockSpec(s, lambda i: (0, 0))
    return pl.pallas_call(
        _encode_body,
        grid=(grid,),
        in_specs=[
            pl.BlockSpec((BN, IN_DIM), lambda i: (i, 0)),
            fullw((IN_DIM, HID)), fullw((1, HID)), fullw((1, HID)),
            fullw((1, HID)), fullw((HID, 2 * HID)),
        ],
        out_specs=[pl.BlockSpec((BN, HID), lambda i: (i, 0)),
                   pl.BlockSpec((BN, 2 * HID), lambda i: (i, 0))],
        out_shape=[jax.ShapeDtypeStruct((NPAD, HID), jnp.float32),
                   jax.ShapeDtypeStruct((NPAD, 2 * HID), jnp.float32)],
    )(x, w_in, b_in, g, b, wab)


def _quantile_body(eat_ref, out_ref):
    v = eat_ref[...]
    r2 = v[0:1] ** 2 + v[1:2] ** 2 + v[2:3] ** 2
    r = jnp.maximum(jnp.sqrt(r2), 1e-8)
    rbits = lax.bitcast_convert_type(r, jnp.int32)
    lo0 = jnp.min(rbits)
    hi0 = jnp.max(rbits)

    def orderstat(k):
        def body(_, carry):
            lo, hi = carry
            mid = lo + (hi - lo) // 2
            midf = lax.bitcast_convert_type(mid, jnp.float32)
            cnt = jnp.sum((r <= midf).astype(jnp.int32))
            ge = cnt >= (k + 1)
            return (jnp.where(ge, lo, mid + 1), jnp.where(ge, mid, hi))
        lo, hi = lax.fori_loop(0, 32, body, (lo0, hi0))
        return lax.bitcast_convert_type(hi, jnp.float32)

    q_pos = 0.95 * (E - 1)
    k_lo = int(q_pos)
    frac = jnp.float32(q_pos - k_lo)
    r1 = orderstat(k_lo)
    r2s = orderstat(k_lo + 1)
    q = r1 * (1.0 - frac) + r2s * frac
    out_ref[0, 0] = jnp.clip(q, 1.0, 8.0)


def _quantile(ea_t):
    return pl.pallas_call(
        _quantile_body,
        in_specs=[pl.BlockSpec((4, E), lambda: (0, 0))],
        out_specs=pl.BlockSpec((1, 1), lambda: (0, 0), memory_space=pltpu.SMEM),
        out_shape=jax.ShapeDtypeStruct((1, 1), jnp.float32),
    )(ea_t)


def _edge_enc_body(eat_ref, rmax_ref, w_e_ref, b_e_ref, g_ref, bb_ref, out_ref):
    v = eat_ref[...]
    r = jnp.maximum(jnp.sqrt(v[0:1] ** 2 + v[1:2] ** 2 + v[2:3] ** 2), 1e-8)
    u = v[0:3] / r
    r_max = rmax_ref[0, 0]
    delta = jnp.maximum(r_max / (RBF_K - 1), 1e-3)
    gamma = 1.0 / (2.0 * (0.5 * delta) ** 2)
    kk = lax.broadcasted_iota(jnp.int32, (RBF_K, BE), 0).astype(jnp.float32)
    centers = r_max * kk / (RBF_K - 1)
    rbf = jnp.exp(-gamma * (r - centers) ** 2)
    e_t = jnp.concatenate([u, r, rbf], axis=0)
    t = lax.dot_general(e_t, w_e_ref[...], (((0,), (0,)), ((), ())),
                        preferred_element_type=jnp.float32) + b_e_ref[...]
    out_ref[...] = _ln(_silu(t), g_ref[...], bb_ref[...])


def _edge_enc(ea_t, r_max, w_e, b_e, g, b):
    fullw = lambda s: pl.BlockSpec(s, lambda i: (0, 0))
    return pl.pallas_call(
        _edge_enc_body,
        grid=(E // BE,),
        in_specs=[
            pl.BlockSpec((4, BE), lambda i: (0, i)),
            pl.BlockSpec(memory_space=pltpu.SMEM),
            fullw((3 + 1 + RBF_K, HID)), fullw((1, HID)),
            fullw((1, HID)), fullw((1, HID)),
        ],
        out_specs=pl.BlockSpec((BE, HID), lambda i: (i, 0)),
        out_shape=jax.ShapeDtypeStruct((E, HID), jnp.float32),
    )(ea_t, r_max, w_e, b_e, g, b)


def _message_body(gath_ref, eenc_ref, w1c_ref, b1_ref, w2_ref, b2t_ref, out_ref):
    pre = gath_ref[...] + jnp.dot(eenc_ref[...], w1c_ref[...],
                                  preferred_element_type=jnp.float32) + b1_ref[...]
    t = _silu(pre)
    m2t = lax.dot_general(w2_ref[...], t, (((0,), (1,)), ((), ())),
                          preferred_element_type=jnp.float32)
    out_ref[...] = _silu(m2t + b2t_ref[...])


def _message(gathered, e_enc, w1c, b1, w2, b2t):
    fullw = lambda s: pl.BlockSpec(s, lambda i: (0, 0))
    return pl.pallas_call(
        _message_body,
        grid=(E // BE,),
        in_specs=[
            pl.BlockSpec((BE, HID), lambda i: (i, 0)),
            pl.BlockSpec((BE, HID), lambda i: (i, 0)),
            fullw((HID, HID)), fullw((1, HID)),
            fullw((HID, HID)), fullw((HID, 1)),
        ],
        out_specs=pl.BlockSpec((HID, BE), lambda i: (0, i)),
        out_shape=jax.ShapeDtypeStruct((HID, E), jnp.float32),
    )(gathered, e_enc, w1c, b1, w2, b2t)


def _update_body(h_ref, ms_ref, wuh_ref, wum_ref, bu_ref, g_ref, bb_ref,
                 wab_ref, h_out, ab_out):
    h = h_ref[...]
    ms = ms_ref[...]
    m_t = ms[0] + ms[1] + ms[2] + ms[3]
    t = (jnp.dot(h, wuh_ref[...], preferred_element_type=jnp.float32)
         + lax.dot_general(m_t, wum_ref[...], (((0,), (0,)), ((), ())),
                           preferred_element_type=jnp.float32)
         + bu_ref[...])
    hn = _ln(_silu(t) + h, g_ref[...], bb_ref[...])
    h_out[...] = hn
    if ab_out is not None:
        ab_out[...] = jnp.dot(hn, wab_ref[...],
                              preferred_element_type=jnp.float32)


def _update(h, msums, wuh, wum, bu, g, b, wab, with_ab):
    fullw = lambda s: pl.BlockSpec(s, lambda i: (0, 0))
    body = _update_body if with_ab else (
        lambda h_ref, ms_ref, wuh_ref, wum_ref, bu_ref, g_ref, bb_ref,
               wab_ref, h_out:
        _update_body(h_ref, ms_ref, wuh_ref, wum_ref, bu_ref, g_ref, bb_ref,
                     wab_ref, h_out, None))
    out_specs = [pl.BlockSpec((BN, HID), lambda i: (i, 0))]
    out_shape = [jax.ShapeDtypeStruct((NPAD, HID), jnp.float32)]
    if with_ab:
        out_specs.append(pl.BlockSpec((BN, 2 * HID), lambda i: (i, 0)))
        out_shape.append(jax.ShapeDtypeStruct((NPAD, 2 * HID), jnp.float32))
    out = pl.pallas_call(
        body,
        grid=(NPAD // BN,),
        in_specs=[
            pl.BlockSpec((BN, HID), lambda i: (i, 0)),
            pl.BlockSpec((4, HID, BN), lambda i: (0, 0, i)),
            fullw((HID, HID)), fullw((HID, HID)), fullw((1, HID)),
            fullw((1, HID)), fullw((1, HID)),
            fullw((HID, 2 * HID)),
        ],
        out_specs=out_specs,
        out_shape=out_shape,
    )(h, msums, wuh, wum, bu, g, b, wab)
    return out if with_ab else (out[0], None)


def _head_body(h0_ref, h1_ref, h2_ref, h3_ref, wjk0, wjk1, wjk2, wjk3,
               bjk_ref, wo1_ref, bo1_ref, wo2_ref, bo2_ref, out_ref):
    agg = (jnp.dot(h0_ref[...], wjk0[...], preferred_element_type=jnp.float32)
           + jnp.dot(h1_ref[...], wjk1[...], preferred_element_type=jnp.float32)
           + jnp.dot(h2_ref[...], wjk2[...], preferred_element_type=jnp.float32)
           + jnp.dot(h3_ref[...], wjk3[...], preferred_element_type=jnp.float32)
           + bjk_ref[...])
    z = _silu(jnp.dot(agg, wo1_ref[...],
                      preferred_element_type=jnp.float32) + bo1_ref[...])
    out_ref[...] = jnp.dot(z, wo2_ref[...],
                           preferred_element_type=jnp.float32) + bo2_ref[...]


def _head(hs, wjks, bjk, wo1, bo1, wo2, bo2):
    fullw = lambda s: pl.BlockSpec(s, lambda i: (0, 0))
    return pl.pallas_call(
        _head_body,
        grid=(NPAD // BN,),
        in_specs=(
            [pl.BlockSpec((BN, HID), lambda i: (i, 0))] * 4
            + [fullw((HID, HID))] * 4
            + [fullw((1, HID)), fullw((HID, HID)), fullw((1, HID)),
               fullw((HID, NCLS)), fullw((1, NCLS))]
        ),
        out_specs=pl.BlockSpec((BN, NCLS), lambda i: (i, 0)),
        out_shape=jax.ShapeDtypeStruct((NPAD, NCLS), jnp.float32),
    )(*hs, *wjks, bjk, wo1, bo1, wo2, bo2)


def _pool_body(xw_ref, lg_ref, out_ref):
    xw = xw_ref[...]
    logits = lg_ref[...]
    rowmax = jnp.max(xw, axis=1, keepdims=True)
    colidx = lax.broadcasted_iota(jnp.int32, (NPAD, NCLS), 1)
    cand = jnp.where(xw == rowmax, colidx, NCLS)
    gid = jnp.min(cand, axis=1, keepdims=True)
    valid = lax.broadcasted_iota(jnp.int32, (NPAD, 1), 0) < N
    onehot = jnp.where((colidx == gid) & valid, 1.0, 0.0)
    sums = lax.dot_general(onehot, logits, (((0,), (0,)), ((), ())),
                           preferred_element_type=jnp.float32)
    counts = jnp.sum(onehot, axis=0, keepdims=True)
    invc = 1.0 / jnp.maximum(counts, 1.0)
    node_ms = jnp.dot(onehot, sums, preferred_element_type=jnp.float32)
    node_inv = jnp.sum(onehot * invc, axis=1, keepdims=True)
    out_ref[...] = node_ms * node_inv


def _pool(xw, logits):
    return pl.pallas_call(
        _pool_body,
        in_specs=[pl.BlockSpec((NPAD, NCLS), lambda: (0, 0))] * 2,
        out_specs=pl.BlockSpec((NPAD, NCLS), lambda: (0, 0)),
        out_shape=jax.ShapeDtypeStruct((NPAD, NCLS), jnp.float32),
    )(xw, logits)


_MESH = dict(core_axis_name="c", subcore_axis_name="s")


def _gather_add_sc(ab, src, dst):
    mesh = plsc.VectorSubcoreMesh(**_MESH)

    @functools.partial(
        pl.kernel,
        out_type=jax.ShapeDtypeStruct((E, HID), jnp.float32),
        mesh=mesh,
        scratch_types=[
            pltpu.VMEM((GC,), jnp.int32),
            pltpu.VMEM((GC,), jnp.int32),
            pltpu.VMEM((GC, 2 * HID), jnp.float32),
            pltpu.VMEM((GC, 2 * HID), jnp.float32),
            pltpu.VMEM((GC, HID), jnp.float32),
            pltpu.SemaphoreType.DMA,
            pltpu.SemaphoreType.DMA,
        ],
    )
    def k(ab_hbm, src_hbm, dst_hbm, out_hbm,
          sidx, didx, arows, brows, obuf, sem1, sem2):
        wid = lax.axis_index("s") * 2 + lax.axis_index("c")

        def chunk(ch, _):
            base = wid * EW + ch * GC
            pltpu.sync_copy(src_hbm.at[pl.ds(base, GC)], sidx)
            pltpu.sync_copy(dst_hbm.at[pl.ds(base, GC)], didx)
            cp_a = pltpu.async_copy(ab_hbm.at[sidx], arows, sem1)
            cp_b = pltpu.async_copy(ab_hbm.at[didx], brows, sem2)
            cp_a.wait()
            cp_b.wait()

            def row(i, _):
                for q in range(HID // 16):
                    sl = pl.ds(q * 16, 16)
                    obuf[i, sl] = arows[i, sl] + brows[i, pl.ds(HID + q * 16, 16)]
                return 0
            lax.fori_loop(0, GC, row, 0)
            pltpu.sync_copy(obuf, out_hbm.at[pl.ds(base, GC)])
            return 0

        lax.fori_loop(0, EW // GC, chunk, 0)

    return k(ab, src, dst)


NG = 4
NFS = 8
EG = E // NG
SCH = 3200


def _scatter_add_sc(m2t, dst):
    mesh = plsc.VectorSubcoreMesh(**_MESH)

    @functools.partial(
        pl.kernel,
        out_type=jax.ShapeDtypeStruct((NG, HID, NPAD), jnp.float32),
        mesh=mesh,
        scratch_types=[
            pltpu.VMEM((SCH,), jnp.int32),
            pltpu.VMEM((NFS, SCH), jnp.float32),
            pltpu.VMEM((NFS * NPAD,), jnp.float32),
        ],
        compiler_params=pltpu.CompilerParams(needs_layout_passes=False),
    )
    def k(m2t_hbm, dst_hbm, out_hbm, didx, rows, acc):
        wid = lax.axis_index("s") * 2 + lax.axis_index("c")
        fs = wid % NFS
        g = wid // NFS

        zero16 = jnp.zeros((16,), jnp.float32)

        def zrow(i, _):
            acc[pl.ds(i * 16, 16)] = zero16
            return 0
        lax.fori_loop(0, NFS * NPAD // 16, zrow, 0)

        def chunk(ch, _):
            base = g * EG + ch * SCH
            pltpu.sync_copy(dst_hbm.at[pl.ds(base, SCH)], didx)
            pltpu.sync_copy(
                m2t_hbm.at[pl.ds(fs * NFS, NFS), pl.ds(base, SCH)], rows)

            full = jnp.ones((16,), jnp.bool_)
            UNR = 4

            def vec(j, _):
                for u in range(UNR):
                    off = j * (16 * UNR) + u * 16
                    dstv = didx[pl.ds(off, 16)]
                    for f in range(NFS):
                        vals = rows[f, pl.ds(off, 16)]
                        plsc.addupdate_scatter(acc, [dstv + f * NPAD], vals,
                                               mask=full)
                return 0
            lax.fori_loop(0, SCH // (16 * UNR), vec, 0)
            return 0
        lax.fori_loop(0, EG // SCH, chunk, 0)

        for f in range(NFS):
            pltpu.sync_copy(acc.at[pl.ds(f * NPAD, NPAD)],
                            out_hbm.at[g, fs * NFS + f])

    return k(m2t, dst)


def kernel(x, edge_index, edge_attr, params):
    p = params
    xp = jnp.pad(x, ((0, NPAD - N), (0, 0)))
    ea_t = edge_attr.T
    src = edge_index[0]
    dst = edge_index[1]

    row = lambda v: v.reshape(1, -1)

    w1ab = [p['msg_W1'][l][:2 * HID] for l in range(LAYERS)]
    w1c = [p['msg_W1'][l][2 * HID:] for l in range(LAYERS)]
    wab = [jnp.concatenate([w[:HID], w[HID:]], axis=1) for w in w1ab]

    h, ab = _encode(xp, p['W_in'], row(p['b_in']), row(p['ln_in_g']),
                    row(p['ln_in_b']), wab[0])

    r_max = _quantile(ea_t)
    e_enc = _edge_enc(ea_t, r_max, p['W_e'], row(p['b_e']),
                      row(p['ln_e_g']), row(p['ln_e_b']))

    states = [h]
    for l in range(LAYERS):
        gathered = _gather_add_sc(ab, src, dst)
        m2 = _message(gathered, e_enc, w1c[l], row(p['msg_b1'][l]),
                      p['msg_W2'][l], p['msg_b2'][l].reshape(-1, 1))
        msums = _scatter_add_sc(m2, dst)
        with_ab = l < LAYERS - 1
        wab_n = wab[l + 1] if with_ab else wab[0]
        h, ab = _update(h, msums, p['upd_W'][l][:HID], p['upd_W'][l][HID:],
                        row(p['upd_b'][l]), row(p['ln_g'][l]),
                        row(p['ln_b'][l]), wab_n, with_ab)
        states.append(h)

    wjks = [p['W_jk'][l * HID:(l + 1) * HID] for l in range(LAYERS + 1)]
    logits = _head(states, wjks, row(p['b_jk']), p['W_o1'], row(p['b_o1']),
                   p['W_o2'], row(p['b_o2']))

    xw = xp[:, WY_START:WY_START + WY_DIM]
    out = _pool(xw, logits)
    return out[:N]


# speedup vs baseline: 3.0928x; 1.0775x over previous
import functools
from typing import Sequence

import jax
import jax.numpy as jnp
from jax import lax
from jax.experimental import pallas as pl
from jax.experimental.pallas import tpu as pltpu
from jax.experimental.pallas import tpu_sc as plsc

N = 10000
NPAD = 10240
E = 320000
IN_DIM = 128
HID = 64
LAYERS = 3
RBF_K = 16
NCLS = 26
WY_START = 100
WY_DIM = 26

BN = 1024
BE = 2560
NW = 32
EW = E // NW
GC = 200
SC_CH = 1000
ROWS_PER_TILE = NPAD // 16


def _silu(x):
    return x * (1.0 / (1.0 + jnp.exp(-x)))


def _ln(x, g, b, eps=1e-5):
    mu = jnp.mean(x, axis=-1, keepdims=True)
    var = jnp.mean((x - mu) ** 2, axis=-1, keepdims=True)
    return (x - mu) / jnp.sqrt(var + eps) * g + b


def _pack_ab(full):
    a16 = lax.bitcast_convert_type(full[:, :HID].astype(jnp.bfloat16),
                                   jnp.uint16).astype(jnp.uint32)
    b16 = lax.bitcast_convert_type(full[:, HID:].astype(jnp.bfloat16),
                                   jnp.uint16).astype(jnp.uint32)
    return lax.bitcast_convert_type(a16 | (b16 << 16), jnp.int32)


def _encode_body(x_ref, w_in_ref, b_in_ref, g_ref, bb_ref, wab_ref,
                 h_ref, ab_ref):
    t = jnp.dot(x_ref[...], w_in_ref[...],
                preferred_element_type=jnp.float32) + b_in_ref[...]
    h = _silu(_ln(t, g_ref[...], bb_ref[...]))
    h_ref[...] = h
    ab_ref[...] = jnp.dot(h, wab_ref[...],
                          preferred_element_type=jnp.float32)


def _encode(x, w_in, b_in, g, b, wab):
    grid = NPAD // BN
    fullw = lambda s: pl.BlockSpec(s, lambda i: (0, 0))
    return pl.pallas_call(
        _encode_body,
        grid=(grid,),
        in_specs=[
            pl.BlockSpec((BN, IN_DIM), lambda i: (i, 0)),
            fullw((IN_DIM, HID)), fullw((1, HID)), fullw((1, HID)),
            fullw((1, HID)), fullw((HID, 2 * HID)),
        ],
        out_specs=[pl.BlockSpec((BN, HID), lambda i: (i, 0)),
                   pl.BlockSpec((BN, 2 * HID), lambda i: (i, 0))],
        out_shape=[jax.ShapeDtypeStruct((NPAD, HID), jnp.float32),
                   jax.ShapeDtypeStruct((NPAD, 2 * HID), jnp.float32)],
    )(x, w_in, b_in, g, b, wab)


def _quantile_body(eat_ref, out_ref):
    v = eat_ref[...]
    r2 = v[0:1] ** 2 + v[1:2] ** 2 + v[2:3] ** 2
    r = jnp.maximum(jnp.sqrt(r2), 1e-8)
    rbits = lax.bitcast_convert_type(r, jnp.int32)
    lo0 = jnp.min(rbits)
    hi0 = jnp.max(rbits)

    def orderstat(k):
        def body(_, carry):
            lo, hi = carry
            mid = lo + (hi - lo) // 2
            midf = lax.bitcast_convert_type(mid, jnp.float32)
            cnt = jnp.sum((r <= midf).astype(jnp.int32))
            ge = cnt >= (k + 1)
            return (jnp.where(ge, lo, mid + 1), jnp.where(ge, mid, hi))
        lo, hi = lax.fori_loop(0, 32, body, (lo0, hi0))
        return lax.bitcast_convert_type(hi, jnp.float32)

    q_pos = 0.95 * (E - 1)
    k_lo = int(q_pos)
    frac = jnp.float32(q_pos - k_lo)
    r1 = orderstat(k_lo)
    r2s = orderstat(k_lo + 1)
    q = r1 * (1.0 - frac) + r2s * frac
    out_ref[0, 0] = jnp.clip(q, 1.0, 8.0)


def _quantile(ea_t):
    return pl.pallas_call(
        _quantile_body,
        in_specs=[pl.BlockSpec((4, E), lambda: (0, 0))],
        out_specs=pl.BlockSpec((1, 1), lambda: (0, 0), memory_space=pltpu.SMEM),
        out_shape=jax.ShapeDtypeStruct((1, 1), jnp.float32),
    )(ea_t)


def _edge_enc_body(eat_ref, rmax_ref, w_e_ref, b_e_ref, g_ref, bb_ref, out_ref):
    v = eat_ref[...]
    r = jnp.maximum(jnp.sqrt(v[0:1] ** 2 + v[1:2] ** 2 + v[2:3] ** 2), 1e-8)
    u = v[0:3] / r
    r_max = rmax_ref[0, 0]
    delta = jnp.maximum(r_max / (RBF_K - 1), 1e-3)
    gamma = 1.0 / (2.0 * (0.5 * delta) ** 2)
    kk = lax.broadcasted_iota(jnp.int32, (RBF_K, BE), 0).astype(jnp.float32)
    centers = r_max * kk / (RBF_K - 1)
    rbf = jnp.exp(-gamma * (r - centers) ** 2)
    e_t = jnp.concatenate([u, r, rbf], axis=0)
    t = lax.dot_general(e_t, w_e_ref[...], (((0,), (0,)), ((), ())),
                        preferred_element_type=jnp.float32) + b_e_ref[...]
    out_ref[...] = _ln(_silu(t), g_ref[...], bb_ref[...])


def _edge_enc(ea_t, r_max, w_e, b_e, g, b):
    fullw = lambda s: pl.BlockSpec(s, lambda i: (0, 0))
    return pl.pallas_call(
        _edge_enc_body,
        grid=(E // BE,),
        in_specs=[
            pl.BlockSpec((4, BE), lambda i: (0, i)),
            pl.BlockSpec(memory_space=pltpu.SMEM),
            fullw((3 + 1 + RBF_K, HID)), fullw((1, HID)),
            fullw((1, HID)), fullw((1, HID)),
        ],
        out_specs=pl.BlockSpec((BE, HID), lambda i: (i, 0)),
        out_shape=jax.ShapeDtypeStruct((E, HID), jnp.float32),
    )(ea_t, r_max, w_e, b_e, g, b)


def _message_body(gath_ref, eenc_ref, w1c_ref, b1_ref, w2_ref, b2t_ref, out_ref):
    pre = gath_ref[...] + jnp.dot(eenc_ref[...], w1c_ref[...],
                                  preferred_element_type=jnp.float32) + b1_ref[...]
    t = _silu(pre)
    m2t = lax.dot_general(w2_ref[...], t, (((0,), (1,)), ((), ())),
                          preferred_element_type=jnp.float32)
    out_ref[...] = _silu(m2t + b2t_ref[...])


def _message(gathered, e_enc, w1c, b1, w2, b2t):
    fullw = lambda s: pl.BlockSpec(s, lambda i: (0, 0))
    return pl.pallas_call(
        _message_body,
        grid=(E // BE,),
        in_specs=[
            pl.BlockSpec((BE, HID), lambda i: (i, 0)),
            pl.BlockSpec((BE, HID), lambda i: (i, 0)),
            fullw((HID, HID)), fullw((1, HID)),
            fullw((HID, HID)), fullw((HID, 1)),
        ],
        out_specs=pl.BlockSpec((HID, BE), lambda i: (0, i)),
        out_shape=jax.ShapeDtypeStruct((HID, E), jnp.float32),
    )(gathered, e_enc, w1c, b1, w2, b2t)


def _update_body(h_ref, ms_ref, wuh_ref, wum_ref, bu_ref, g_ref, bb_ref,
                 wab_ref, h_out, ab_out):
    h = h_ref[...]
    ms = ms_ref[...]
    m_t = ms[0] + ms[1] + ms[2] + ms[3]
    t = (jnp.dot(h, wuh_ref[...], preferred_element_type=jnp.float32)
         + lax.dot_general(m_t, wum_ref[...], (((0,), (0,)), ((), ())),
                           preferred_element_type=jnp.float32)
         + bu_ref[...])
    hn = _ln(_silu(t) + h, g_ref[...], bb_ref[...])
    h_out[...] = hn
    if ab_out is not None:
        ab_out[...] = jnp.dot(hn, wab_ref[...],
                              preferred_element_type=jnp.float32)


def _update(h, msums, wuh, wum, bu, g, b, wab, with_ab):
    fullw = lambda s: pl.BlockSpec(s, lambda i: (0, 0))
    body = _update_body if with_ab else (
        lambda h_ref, ms_ref, wuh_ref, wum_ref, bu_ref, g_ref, bb_ref,
               wab_ref, h_out:
        _update_body(h_ref, ms_ref, wuh_ref, wum_ref, bu_ref, g_ref, bb_ref,
                     wab_ref, h_out, None))
    out_specs = [pl.BlockSpec((BN, HID), lambda i: (i, 0))]
    out_shape = [jax.ShapeDtypeStruct((NPAD, HID), jnp.float32)]
    if with_ab:
        out_specs.append(pl.BlockSpec((BN, 2 * HID), lambda i: (i, 0)))
        out_shape.append(jax.ShapeDtypeStruct((NPAD, 2 * HID), jnp.float32))
    out = pl.pallas_call(
        body,
        grid=(NPAD // BN,),
        in_specs=[
            pl.BlockSpec((BN, HID), lambda i: (i, 0)),
            pl.BlockSpec((4, HID, BN), lambda i: (0, 0, i)),
            fullw((HID, HID)), fullw((HID, HID)), fullw((1, HID)),
            fullw((1, HID)), fullw((1, HID)),
            fullw((HID, 2 * HID)),
        ],
        out_specs=out_specs,
        out_shape=out_shape,
    )(h, msums, wuh, wum, bu, g, b, wab)
    return out if with_ab else (out[0], None)


def _head_body(h0_ref, h1_ref, h2_ref, h3_ref, wjk0, wjk1, wjk2, wjk3,
               bjk_ref, wo1_ref, bo1_ref, wo2_ref, bo2_ref, out_ref):
    agg = (jnp.dot(h0_ref[...], wjk0[...], preferred_element_type=jnp.float32)
           + jnp.dot(h1_ref[...], wjk1[...], preferred_element_type=jnp.float32)
           + jnp.dot(h2_ref[...], wjk2[...], preferred_element_type=jnp.float32)
           + jnp.dot(h3_ref[...], wjk3[...], preferred_element_type=jnp.float32)
           + bjk_ref[...])
    z = _silu(jnp.dot(agg, wo1_ref[...],
                      preferred_element_type=jnp.float32) + bo1_ref[...])
    out_ref[...] = jnp.dot(z, wo2_ref[...],
                           preferred_element_type=jnp.float32) + bo2_ref[...]


def _head(hs, wjks, bjk, wo1, bo1, wo2, bo2):
    fullw = lambda s: pl.BlockSpec(s, lambda i: (0, 0))
    return pl.pallas_call(
        _head_body,
        grid=(NPAD // BN,),
        in_specs=(
            [pl.BlockSpec((BN, HID), lambda i: (i, 0))] * 4
            + [fullw((HID, HID))] * 4
            + [fullw((1, HID)), fullw((HID, HID)), fullw((1, HID)),
               fullw((HID, NCLS)), fullw((1, NCLS))]
        ),
        out_specs=pl.BlockSpec((BN, NCLS), lambda i: (i, 0)),
        out_shape=jax.ShapeDtypeStruct((NPAD, NCLS), jnp.float32),
    )(*hs, *wjks, bjk, wo1, bo1, wo2, bo2)


def _pool_body(xw_ref, lg_ref, out_ref):
    xw = xw_ref[...]
    logits = lg_ref[...]
    rowmax = jnp.max(xw, axis=1, keepdims=True)
    colidx = lax.broadcasted_iota(jnp.int32, (NPAD, NCLS), 1)
    cand = jnp.where(xw == rowmax, colidx, NCLS)
    gid = jnp.min(cand, axis=1, keepdims=True)
    valid = lax.broadcasted_iota(jnp.int32, (NPAD, 1), 0) < N
    onehot = jnp.where((colidx == gid) & valid, 1.0, 0.0)
    sums = lax.dot_general(onehot, logits, (((0,), (0,)), ((), ())),
                           preferred_element_type=jnp.float32)
    counts = jnp.sum(onehot, axis=0, keepdims=True)
    invc = 1.0 / jnp.maximum(counts, 1.0)
    node_ms = jnp.dot(onehot, sums, preferred_element_type=jnp.float32)
    node_inv = jnp.sum(onehot * invc, axis=1, keepdims=True)
    out_ref[...] = node_ms * node_inv


def _pool(xw, logits):
    return pl.pallas_call(
        _pool_body,
        in_specs=[pl.BlockSpec((NPAD, NCLS), lambda: (0, 0))] * 2,
        out_specs=pl.BlockSpec((NPAD, NCLS), lambda: (0, 0)),
        out_shape=jax.ShapeDtypeStruct((NPAD, NCLS), jnp.float32),
    )(xw, logits)


_MESH = dict(core_axis_name="c", subcore_axis_name="s")


def _gather_add_sc(ab, src, dst):
    mesh = plsc.VectorSubcoreMesh(**_MESH)

    @functools.partial(
        pl.kernel,
        out_type=jax.ShapeDtypeStruct((E, HID), jnp.float32),
        mesh=mesh,
        scratch_types=[
            pltpu.VMEM((GC,), jnp.int32),
            pltpu.VMEM((GC,), jnp.int32),
            pltpu.VMEM((GC,), jnp.int32),
            pltpu.VMEM((GC,), jnp.int32),
            pltpu.VMEM((GC, 2 * HID), jnp.float32),
            pltpu.VMEM((GC, 2 * HID), jnp.float32),
            pltpu.VMEM((GC, 2 * HID), jnp.float32),
            pltpu.VMEM((GC, 2 * HID), jnp.float32),
            pltpu.VMEM((GC, HID), jnp.float32),
            pltpu.SemaphoreType.DMA,
            pltpu.SemaphoreType.DMA,
            pltpu.SemaphoreType.DMA,
            pltpu.SemaphoreType.DMA,
        ],
        compiler_params=pltpu.CompilerParams(needs_layout_passes=False),
    )
    def k(ab_hbm, src_hbm, dst_hbm, out_hbm,
          sidx0, sidx1, didx0, didx1, arows0, brows0, arows1, brows1, obuf,
          sa0, sb0, sa1, sb1):
        wid = lax.axis_index("s") * 2 + lax.axis_index("c")
        NCH = EW // GC
        sbufs = (sidx0, sidx1)
        dbufs = (didx0, didx1)
        abuf = (arows0, arows1)
        bbuf = (brows0, brows1)
        asem = (sa0, sa1)
        bsem = (sb0, sb1)

        def issue(ch, s):
            base = wid * EW + ch * GC
            pltpu.sync_copy(src_hbm.at[pl.ds(base, GC)], sbufs[s])
            pltpu.sync_copy(dst_hbm.at[pl.ds(base, GC)], dbufs[s])
            pltpu.async_copy(ab_hbm.at[sbufs[s]], abuf[s], asem[s])
            pltpu.async_copy(ab_hbm.at[dbufs[s]], bbuf[s], bsem[s])

        def consume(ch, s):
            pltpu.make_async_copy(ab_hbm.at[sbufs[s]], abuf[s],
                                  asem[s]).wait()
            pltpu.make_async_copy(ab_hbm.at[dbufs[s]], bbuf[s],
                                  bsem[s]).wait()
            ar, br = abuf[s], bbuf[s]

            def row(i, _):
                for q in range(HID // 16):
                    sl = pl.ds(q * 16, 16)
                    obuf[i, sl] = ar[i, sl] + br[i, pl.ds(HID + q * 16, 16)]
                return 0
            lax.fori_loop(0, GC, row, 0)
            base = wid * EW + ch * GC
            pltpu.sync_copy(obuf, out_hbm.at[pl.ds(base, GC)])

        issue(0, 0)

        def pair(ch, _):
            for s in range(2):
                cur = ch + s

                @pl.when(cur + 1 < NCH)
                def _():
                    issue(cur + 1, 1 - s)
                consume(cur, s)
            return 0
        lax.fori_loop(0, NCH // 2, lambda p, c: pair(p * 2, c), 0)

    return k(ab, src, dst)


NG = 4
NFS = 8
EG = E // NG
SCH = 3200


def _scatter_add_sc(m2t, dst):
    mesh = plsc.VectorSubcoreMesh(**_MESH)

    @functools.partial(
        pl.kernel,
        out_type=jax.ShapeDtypeStruct((NG, HID, NPAD), jnp.float32),
        mesh=mesh,
        scratch_types=[
            pltpu.VMEM((SCH,), jnp.int32),
            pltpu.VMEM((NFS, SCH), jnp.float32),
            pltpu.VMEM((NFS * NPAD,), jnp.float32),
        ],
        compiler_params=pltpu.CompilerParams(needs_layout_passes=False),
    )
    def k(m2t_hbm, dst_hbm, out_hbm, didx, rows, acc):
        wid = lax.axis_index("s") * 2 + lax.axis_index("c")
        fs = wid % NFS
        g = wid // NFS

        zero16 = jnp.zeros((16,), jnp.float32)

        def zrow(i, _):
            acc[pl.ds(i * 16, 16)] = zero16
            return 0
        lax.fori_loop(0, NFS * NPAD // 16, zrow, 0)

        def chunk(ch, _):
            base = g * EG + ch * SCH
            pltpu.sync_copy(dst_hbm.at[pl.ds(base, SCH)], didx)
            pltpu.sync_copy(
                m2t_hbm.at[pl.ds(fs * NFS, NFS), pl.ds(base, SCH)], rows)

            full = jnp.ones((16,), jnp.bool_)

            def vec(j, _):
                dstv = didx[pl.ds(j * 16, 16)]
                for f in range(NFS):
                    vals = rows[f, pl.ds(j * 16, 16)]
                    plsc.addupdate_scatter(acc, [dstv + f * NPAD], vals,
                                           mask=full)
                return 0
            lax.fori_loop(0, SCH // 16, vec, 0)
            return 0
        lax.fori_loop(0, EG // SCH, chunk, 0)

        for f in range(NFS):
            pltpu.sync_copy(acc.at[pl.ds(f * NPAD, NPAD)],
                            out_hbm.at[g, fs * NFS + f])

    return k(m2t, dst)


def kernel(x, edge_index, edge_attr, params):
    p = params
    xp = jnp.pad(x, ((0, NPAD - N), (0, 0)))
    ea_t = edge_attr.T
    src = edge_index[0]
    dst = edge_index[1]

    row = lambda v: v.reshape(1, -1)

    w1ab = [p['msg_W1'][l][:2 * HID] for l in range(LAYERS)]
    w1c = [p['msg_W1'][l][2 * HID:] for l in range(LAYERS)]
    wab = [jnp.concatenate([w[:HID], w[HID:]], axis=1) for w in w1ab]

    h, ab = _encode(xp, p['W_in'], row(p['b_in']), row(p['ln_in_g']),
                    row(p['ln_in_b']), wab[0])

    r_max = _quantile(ea_t)
    e_enc = _edge_enc(ea_t, r_max, p['W_e'], row(p['b_e']),
                      row(p['ln_e_g']), row(p['ln_e_b']))

    states = [h]
    for l in range(LAYERS):
        gathered = _gather_add_sc(ab, src, dst)
        m2 = _message(gathered, e_enc, w1c[l], row(p['msg_b1'][l]),
                      p['msg_W2'][l], p['msg_b2'][l].reshape(-1, 1))
        msums = _scatter_add_sc(m2, dst)
        with_ab = l < LAYERS - 1
        wab_n = wab[l + 1] if with_ab else wab[0]
        h, ab = _update(h, msums, p['upd_W'][l][:HID], p['upd_W'][l][HID:],
                        row(p['upd_b'][l]), row(p['ln_g'][l]),
                        row(p['ln_b'][l]), wab_n, with_ab)
        states.append(h)

    wjks = [p['W_jk'][l * HID:(l + 1) * HID] for l in range(LAYERS + 1)]
    logits = _head(states, wjks, row(p['b_jk']), p['W_o1'], row(p['b_o1']),
                   p['W_o2'], row(p['b_o2']))

    xw = xp[:, WY_START:WY_START + WY_DIM]
    out = _pool(xw, logits)
    return out[:N]


# speedup vs baseline: 3.4171x; 1.1048x over previous
import functools
from typing import Sequence

import jax
import jax.numpy as jnp
from jax import lax
from jax.experimental import pallas as pl
from jax.experimental.pallas import tpu as pltpu
from jax.experimental.pallas import tpu_sc as plsc

N = 10000
NPAD = 10240
E = 320000
IN_DIM = 128
HID = 64
LAYERS = 3
RBF_K = 16
NCLS = 26
WY_START = 100
WY_DIM = 26

BN = 1024
BE = 2560
NW = 32
EW = E // NW
GC = 200
SC_CH = 1000
ROWS_PER_TILE = NPAD // 16


def _silu(x):
    return x * (1.0 / (1.0 + jnp.exp(-x)))


def _ln(x, g, b, eps=1e-5):
    mu = jnp.mean(x, axis=-1, keepdims=True)
    var = jnp.mean((x - mu) ** 2, axis=-1, keepdims=True)
    return (x - mu) / jnp.sqrt(var + eps) * g + b


def _pack_ab(full):
    a16 = lax.bitcast_convert_type(full[:, :HID].astype(jnp.bfloat16),
                                   jnp.uint16).astype(jnp.uint32)
    b16 = lax.bitcast_convert_type(full[:, HID:].astype(jnp.bfloat16),
                                   jnp.uint16).astype(jnp.uint32)
    return lax.bitcast_convert_type(a16 | (b16 << 16), jnp.int32)


def _encode_body(x_ref, w_in_ref, b_in_ref, g_ref, bb_ref, wab_ref,
                 h_ref, ab_ref):
    t = jnp.dot(x_ref[...], w_in_ref[...],
                preferred_element_type=jnp.float32) + b_in_ref[...]
    h = _silu(_ln(t, g_ref[...], bb_ref[...]))
    h_ref[...] = h
    ab_ref[...] = jnp.dot(h, wab_ref[...],
                          preferred_element_type=jnp.float32)


def _encode(x, w_in, b_in, g, b, wab):
    grid = NPAD // BN
    fullw = lambda s: pl.BlockSpec(s, lambda i: (0, 0))
    return pl.pallas_call(
        _encode_body,
        grid=(grid,),
        in_specs=[
            pl.BlockSpec((BN, IN_DIM), lambda i: (i, 0)),
            fullw((IN_DIM, HID)), fullw((1, HID)), fullw((1, HID)),
            fullw((1, HID)), fullw((HID, 2 * HID)),
        ],
        out_specs=[pl.BlockSpec((BN, HID), lambda i: (i, 0)),
                   pl.BlockSpec((BN, 2 * HID), lambda i: (i, 0))],
        out_shape=[jax.ShapeDtypeStruct((NPAD, HID), jnp.float32),
                   jax.ShapeDtypeStruct((NPAD, 2 * HID), jnp.float32)],
    )(x, w_in, b_in, g, b, wab)


def _quantile_body(eat_ref, out_ref):
    v = eat_ref[...]
    r2 = v[0:1] ** 2 + v[1:2] ** 2 + v[2:3] ** 2
    r = jnp.maximum(jnp.sqrt(r2), 1e-8)
    rbits = lax.bitcast_convert_type(r, jnp.int32)
    lo0 = jnp.min(rbits)
    hi0 = jnp.max(rbits)

    def orderstat(k):
        def body(_, carry):
            lo, hi = carry
            mid = lo + (hi - lo) // 2
            midf = lax.bitcast_convert_type(mid, jnp.float32)
            cnt = jnp.sum((r <= midf).astype(jnp.int32))
            ge = cnt >= (k + 1)
            return (jnp.where(ge, lo, mid + 1), jnp.where(ge, mid, hi))
        lo, hi = lax.fori_loop(0, 32, body, (lo0, hi0))
        return lax.bitcast_convert_type(hi, jnp.float32)

    q_pos = 0.95 * (E - 1)
    k_lo = int(q_pos)
    frac = jnp.float32(q_pos - k_lo)
    r1 = orderstat(k_lo)
    r2s = orderstat(k_lo + 1)
    q = r1 * (1.0 - frac) + r2s * frac
    out_ref[0, 0] = jnp.clip(q, 1.0, 8.0)


def _quantile(ea_t):
    return pl.pallas_call(
        _quantile_body,
        in_specs=[pl.BlockSpec((4, E), lambda: (0, 0))],
        out_specs=pl.BlockSpec((1, 1), lambda: (0, 0), memory_space=pltpu.SMEM),
        out_shape=jax.ShapeDtypeStruct((1, 1), jnp.float32),
    )(ea_t)


def _edge_enc_body(eat_ref, rmax_ref, w_e_ref, b_e_ref, g_ref, bb_ref, out_ref):
    v = eat_ref[...]
    r = jnp.maximum(jnp.sqrt(v[0:1] ** 2 + v[1:2] ** 2 + v[2:3] ** 2), 1e-8)
    u = v[0:3] / r
    r_max = rmax_ref[0, 0]
    delta = jnp.maximum(r_max / (RBF_K - 1), 1e-3)
    gamma = 1.0 / (2.0 * (0.5 * delta) ** 2)
    kk = lax.broadcasted_iota(jnp.int32, (RBF_K, BE), 0).astype(jnp.float32)
    centers = r_max * kk / (RBF_K - 1)
    rbf = jnp.exp(-gamma * (r - centers) ** 2)
    e_t = jnp.concatenate([u, r, rbf], axis=0)
    t = lax.dot_general(e_t, w_e_ref[...], (((0,), (0,)), ((), ())),
                        preferred_element_type=jnp.float32) + b_e_ref[...]
    out_ref[...] = _ln(_silu(t), g_ref[...], bb_ref[...]).astype(jnp.bfloat16)


def _edge_enc(ea_t, r_max, w_e, b_e, g, b):
    fullw = lambda s: pl.BlockSpec(s, lambda i: (0, 0))
    return pl.pallas_call(
        _edge_enc_body,
        grid=(E // BE,),
        in_specs=[
            pl.BlockSpec((4, BE), lambda i: (0, i)),
            pl.BlockSpec(memory_space=pltpu.SMEM),
            fullw((3 + 1 + RBF_K, HID)), fullw((1, HID)),
            fullw((1, HID)), fullw((1, HID)),
        ],
        out_specs=pl.BlockSpec((BE, HID), lambda i: (i, 0)),
        out_shape=jax.ShapeDtypeStruct((E, HID), jnp.bfloat16),
    )(ea_t, r_max, w_e, b_e, g, b)


def _message_body(gath_ref, eenc_ref, w1c_ref, b1_ref, w2_ref, b2t_ref, out_ref):
    pre = (gath_ref[...]
           + jnp.dot(eenc_ref[...].astype(jnp.float32), w1c_ref[...],
                     preferred_element_type=jnp.float32) + b1_ref[...])
    t = _silu(pre)
    m2t = lax.dot_general(w2_ref[...], t, (((0,), (1,)), ((), ())),
                          preferred_element_type=jnp.float32)
    out_ref[...] = _silu(m2t + b2t_ref[...])


def _message(gathered, e_enc, w1c, b1, w2, b2t):
    fullw = lambda s: pl.BlockSpec(s, lambda i: (0, 0))
    return pl.pallas_call(
        _message_body,
        grid=(E // BE,),
        in_specs=[
            pl.BlockSpec((BE, HID), lambda i: (i, 0)),
            pl.BlockSpec((BE, HID), lambda i: (i, 0)),
            fullw((HID, HID)), fullw((1, HID)),
            fullw((HID, HID)), fullw((HID, 1)),
        ],
        out_specs=pl.BlockSpec((HID, BE), lambda i: (0, i)),
        out_shape=jax.ShapeDtypeStruct((HID, E), jnp.float32),
    )(gathered, e_enc, w1c, b1, w2, b2t)


def _update_body(h_ref, ms_ref, wuh_ref, wum_ref, bu_ref, g_ref, bb_ref,
                 wab_ref, h_out, ab_out):
    h = h_ref[...]
    ms = ms_ref[...]
    m_t = ms[0] + ms[1] + ms[2] + ms[3]
    t = (jnp.dot(h, wuh_ref[...], preferred_element_type=jnp.float32)
         + lax.dot_general(m_t, wum_ref[...], (((0,), (0,)), ((), ())),
                           preferred_element_type=jnp.float32)
         + bu_ref[...])
    hn = _ln(_silu(t) + h, g_ref[...], bb_ref[...])
    h_out[...] = hn
    if ab_out is not None:
        ab_out[...] = jnp.dot(hn, wab_ref[...],
                              preferred_element_type=jnp.float32)


def _update(h, msums, wuh, wum, bu, g, b, wab, with_ab):
    fullw = lambda s: pl.BlockSpec(s, lambda i: (0, 0))
    body = _update_body if with_ab else (
        lambda h_ref, ms_ref, wuh_ref, wum_ref, bu_ref, g_ref, bb_ref,
               wab_ref, h_out:
        _update_body(h_ref, ms_ref, wuh_ref, wum_ref, bu_ref, g_ref, bb_ref,
                     wab_ref, h_out, None))
    out_specs = [pl.BlockSpec((BN, HID), lambda i: (i, 0))]
    out_shape = [jax.ShapeDtypeStruct((NPAD, HID), jnp.float32)]
    if with_ab:
        out_specs.append(pl.BlockSpec((BN, 2 * HID), lambda i: (i, 0)))
        out_shape.append(jax.ShapeDtypeStruct((NPAD, 2 * HID), jnp.float32))
    out = pl.pallas_call(
        body,
        grid=(NPAD // BN,),
        in_specs=[
            pl.BlockSpec((BN, HID), lambda i: (i, 0)),
            pl.BlockSpec((4, HID, BN), lambda i: (0, 0, i)),
            fullw((HID, HID)), fullw((HID, HID)), fullw((1, HID)),
            fullw((1, HID)), fullw((1, HID)),
            fullw((HID, 2 * HID)),
        ],
        out_specs=out_specs,
        out_shape=out_shape,
    )(h, msums, wuh, wum, bu, g, b, wab)
    return out if with_ab else (out[0], None)


def _head_body(h0_ref, h1_ref, h2_ref, h3_ref, wjk0, wjk1, wjk2, wjk3,
               bjk_ref, wo1_ref, bo1_ref, wo2_ref, bo2_ref, out_ref):
    agg = (jnp.dot(h0_ref[...], wjk0[...], preferred_element_type=jnp.float32)
           + jnp.dot(h1_ref[...], wjk1[...], preferred_element_type=jnp.float32)
           + jnp.dot(h2_ref[...], wjk2[...], preferred_element_type=jnp.float32)
           + jnp.dot(h3_ref[...], wjk3[...], preferred_element_type=jnp.float32)
           + bjk_ref[...])
    z = _silu(jnp.dot(agg, wo1_ref[...],
                      preferred_element_type=jnp.float32) + bo1_ref[...])
    out_ref[...] = jnp.dot(z, wo2_ref[...],
                           preferred_element_type=jnp.float32) + bo2_ref[...]


def _head(hs, wjks, bjk, wo1, bo1, wo2, bo2):
    fullw = lambda s: pl.BlockSpec(s, lambda i: (0, 0))
    return pl.pallas_call(
        _head_body,
        grid=(NPAD // BN,),
        in_specs=(
            [pl.BlockSpec((BN, HID), lambda i: (i, 0))] * 4
            + [fullw((HID, HID))] * 4
            + [fullw((1, HID)), fullw((HID, HID)), fullw((1, HID)),
               fullw((HID, NCLS)), fullw((1, NCLS))]
        ),
        out_specs=pl.BlockSpec((BN, NCLS), lambda i: (i, 0)),
        out_shape=jax.ShapeDtypeStruct((NPAD, NCLS), jnp.float32),
    )(*hs, *wjks, bjk, wo1, bo1, wo2, bo2)


def _pool_body(xw_ref, lg_ref, out_ref):
    xw = xw_ref[...]
    logits = lg_ref[...]
    rowmax = jnp.max(xw, axis=1, keepdims=True)
    colidx = lax.broadcasted_iota(jnp.int32, (NPAD, NCLS), 1)
    cand = jnp.where(xw == rowmax, colidx, NCLS)
    gid = jnp.min(cand, axis=1, keepdims=True)
    valid = lax.broadcasted_iota(jnp.int32, (NPAD, 1), 0) < N
    onehot = jnp.where((colidx == gid) & valid, 1.0, 0.0)
    sums = lax.dot_general(onehot, logits, (((0,), (0,)), ((), ())),
                           preferred_element_type=jnp.float32)
    counts = jnp.sum(onehot, axis=0, keepdims=True)
    invc = 1.0 / jnp.maximum(counts, 1.0)
    node_ms = jnp.dot(onehot, sums, preferred_element_type=jnp.float32)
    node_inv = jnp.sum(onehot * invc, axis=1, keepdims=True)
    out_ref[...] = node_ms * node_inv


def _pool(xw, logits):
    return pl.pallas_call(
        _pool_body,
        in_specs=[pl.BlockSpec((NPAD, NCLS), lambda: (0, 0))] * 2,
        out_specs=pl.BlockSpec((NPAD, NCLS), lambda: (0, 0)),
        out_shape=jax.ShapeDtypeStruct((NPAD, NCLS), jnp.float32),
    )(xw, logits)


_MESH = dict(core_axis_name="c", subcore_axis_name="s")


def _gather_add_sc(ab, src, dst):
    mesh = plsc.VectorSubcoreMesh(**_MESH)

    @functools.partial(
        pl.kernel,
        out_type=jax.ShapeDtypeStruct((E, HID), jnp.float32),
        mesh=mesh,
        scratch_types=[
            pltpu.VMEM((GC,), jnp.int32),
            pltpu.VMEM((GC,), jnp.int32),
            pltpu.VMEM((GC,), jnp.int32),
            pltpu.VMEM((GC,), jnp.int32),
            pltpu.VMEM((GC, 2 * HID), jnp.float32),
            pltpu.VMEM((GC, 2 * HID), jnp.float32),
            pltpu.VMEM((GC, 2 * HID), jnp.float32),
            pltpu.VMEM((GC, 2 * HID), jnp.float32),
            pltpu.VMEM((GC, HID), jnp.float32),
            pltpu.SemaphoreType.DMA,
            pltpu.SemaphoreType.DMA,
            pltpu.SemaphoreType.DMA,
            pltpu.SemaphoreType.DMA,
        ],
        compiler_params=pltpu.CompilerParams(needs_layout_passes=False),
    )
    def k(ab_hbm, src_hbm, dst_hbm, out_hbm,
          sidx0, sidx1, didx0, didx1, arows0, brows0, arows1, brows1, obuf,
          sa0, sb0, sa1, sb1):
        wid = lax.axis_index("s") * 2 + lax.axis_index("c")
        NCH = EW // GC
        sbufs = (sidx0, sidx1)
        dbufs = (didx0, didx1)
        abuf = (arows0, arows1)
        bbuf = (brows0, brows1)
        asem = (sa0, sa1)
        bsem = (sb0, sb1)

        def issue(ch, s):
            base = wid * EW + ch * GC
            pltpu.sync_copy(src_hbm.at[pl.ds(base, GC)], sbufs[s])
            pltpu.sync_copy(dst_hbm.at[pl.ds(base, GC)], dbufs[s])
            pltpu.async_copy(ab_hbm.at[sbufs[s]], abuf[s], asem[s])
            pltpu.async_copy(ab_hbm.at[dbufs[s]], bbuf[s], bsem[s])

        def consume(ch, s):
            pltpu.make_async_copy(ab_hbm.at[sbufs[s]], abuf[s],
                                  asem[s]).wait()
            pltpu.make_async_copy(ab_hbm.at[dbufs[s]], bbuf[s],
                                  bsem[s]).wait()
            ar, br = abuf[s], bbuf[s]

            def row(i, _):
                for q in range(HID // 16):
                    sl = pl.ds(q * 16, 16)
                    obuf[i, sl] = ar[i, sl] + br[i, pl.ds(HID + q * 16, 16)]
                return 0
            lax.fori_loop(0, GC, row, 0)
            base = wid * EW + ch * GC
            pltpu.sync_copy(obuf, out_hbm.at[pl.ds(base, GC)])

        issue(0, 0)

        def pair(ch, _):
            for s in range(2):
                cur = ch + s

                @pl.when(cur + 1 < NCH)
                def _():
                    issue(cur + 1, 1 - s)
                consume(cur, s)
            return 0
        lax.fori_loop(0, NCH // 2, lambda p, c: pair(p * 2, c), 0)

    return k(ab, src, dst)


NG = 4
NFS = 8
EG = E // NG
SCH = 640


def _scatter_add_sc(m2t, dst):
    mesh = plsc.VectorSubcoreMesh(**_MESH)

    @functools.partial(
        pl.kernel,
        out_type=jax.ShapeDtypeStruct((NG, HID, NPAD), jnp.float32),
        mesh=mesh,
        scratch_types=[
            pltpu.VMEM((SCH,), jnp.int32),
            pltpu.VMEM((SCH,), jnp.int32),
            pltpu.VMEM((NFS, SCH), jnp.float32),
            pltpu.VMEM((NFS, SCH), jnp.float32),
            pltpu.VMEM((NFS * NPAD,), jnp.float32),
            pltpu.SemaphoreType.DMA,
            pltpu.SemaphoreType.DMA,
            pltpu.SemaphoreType.DMA,
            pltpu.SemaphoreType.DMA,
        ],
        compiler_params=pltpu.CompilerParams(needs_layout_passes=False),
    )
    def k(m2t_hbm, dst_hbm, out_hbm, didx0, didx1, rows0, rows1, acc,
          sd0, sd1, sr0, sr1):
        wid = lax.axis_index("s") * 2 + lax.axis_index("c")
        fs = wid % NFS
        g = wid // NFS
        NCH = EG // SCH
        dbuf = (didx0, didx1)
        rbuf = (rows0, rows1)
        dsem = (sd0, sd1)
        rsem = (sr0, sr1)

        zero16 = jnp.zeros((16,), jnp.float32)

        def zrow(i, _):
            acc[pl.ds(i * 16, 16)] = zero16
            return 0
        lax.fori_loop(0, NFS * NPAD // 16, zrow, 0)

        def issue(ch, s):
            base = g * EG + ch * SCH
            pltpu.async_copy(dst_hbm.at[pl.ds(base, SCH)], dbuf[s], dsem[s])
            pltpu.async_copy(
                m2t_hbm.at[pl.ds(fs * NFS, NFS), pl.ds(base, SCH)],
                rbuf[s], rsem[s])

        def consume(ch, s):
            base = g * EG + ch * SCH
            pltpu.make_async_copy(dst_hbm.at[pl.ds(base, SCH)], dbuf[s],
                                  dsem[s]).wait()
            pltpu.make_async_copy(
                m2t_hbm.at[pl.ds(fs * NFS, NFS), pl.ds(base, SCH)],
                rbuf[s], rsem[s]).wait()
            didx, rows = dbuf[s], rbuf[s]

            full = jnp.ones((16,), jnp.bool_)

            def vec(j, _):
                dstv = didx[pl.ds(j * 16, 16)]
                for f in range(NFS):
                    vals = rows[f, pl.ds(j * 16, 16)]
                    plsc.addupdate_scatter(acc, [dstv + f * NPAD], vals,
                                           mask=full)
                return 0
            lax.fori_loop(0, SCH // 16, vec, 0)

        issue(0, 0)

        def pair(ch, _):
            for s in range(2):
                cur = ch + s

                @pl.when(cur + 1 < NCH)
                def _():
                    issue(cur + 1, 1 - s)
                consume(cur, s)
            return 0
        lax.fori_loop(0, NCH // 2, lambda p, c: pair(p * 2, c), 0)
        consume(NCH - 1, 0)

        for f in range(NFS):
            pltpu.sync_copy(acc.at[pl.ds(f * NPAD, NPAD)],
                            out_hbm.at[g, fs * NFS + f])

    return k(m2t, dst)


def kernel(x, edge_index, edge_attr, params):
    p = params
    xp = jnp.pad(x, ((0, NPAD - N), (0, 0)))
    ea_t = edge_attr.T
    src = edge_index[0]
    dst = edge_index[1]

    row = lambda v: v.reshape(1, -1)

    w1ab = [p['msg_W1'][l][:2 * HID] for l in range(LAYERS)]
    w1c = [p['msg_W1'][l][2 * HID:] for l in range(LAYERS)]
    wab = [jnp.concatenate([w[:HID], w[HID:]], axis=1) for w in w1ab]

    h, ab = _encode(xp, p['W_in'], row(p['b_in']), row(p['ln_in_g']),
                    row(p['ln_in_b']), wab[0])

    r_max = _quantile(ea_t)
    e_enc = _edge_enc(ea_t, r_max, p['W_e'], row(p['b_e']),
                      row(p['ln_e_g']), row(p['ln_e_b']))

    states = [h]
    for l in range(LAYERS):
        gathered = _gather_add_sc(ab, src, dst)
        m2 = _message(gathered, e_enc, w1c[l], row(p['msg_b1'][l]),
                      p['msg_W2'][l], p['msg_b2'][l].reshape(-1, 1))
        msums = _scatter_add_sc(m2, dst)
        with_ab = l < LAYERS - 1
        wab_n = wab[l + 1] if with_ab else wab[0]
        h, ab = _update(h, msums, p['upd_W'][l][:HID], p['upd_W'][l][HID:],
                        row(p['upd_b'][l]), row(p['ln_g'][l]),
                        row(p['ln_b'][l]), wab_n, with_ab)
        states.append(h)

    wjks = [p['W_jk'][l * HID:(l + 1) * HID] for l in range(LAYERS + 1)]
    logits = _head(states, wjks, row(p['b_jk']), p['W_o1'], row(p['b_o1']),
                   p['W_o2'], row(p['b_o2']))

    xw = xp[:, WY_START:WY_START + WY_DIM]
    out = _pool(xw, logits)
    return out[:N]
